# baseline (speedup 1.0000x reference)
import dataclasses
import functools

import jax
import jax.numpy as jnp
from jax import lax
from jax.experimental import pallas as pl
from jax.experimental.pallas import tpu as pltpu
from jax.experimental.pallas import tpu_sc as plsc

NUM_CLASSES = 256
FEAT = 512
Q_BLOCK = 4096

SC_CORES = 2
SC_SUBCORES = 16
SC_TILES = SC_CORES * SC_SUBCORES
LANES = 16
CHUNK = 64
CLS_PER_TILE = NUM_CLASSES // SC_TILES
FEAT_CHUNKS = FEAT // LANES


def _lbl(labels_v, i):
    return labels_v[pl.ds(i, LANES)][0]


def _bsearch_ge(labels_v, target, n):
    pos = jnp.int32(0)
    step = n
    while step > 1:
        step //= 2
        nxt = pos + step
        pos = jnp.where(_lbl(labels_v, nxt - 1) < target, nxt, pos)
    nxt = pos + 1
    pos = jnp.where(_lbl(labels_v, pos) < target, nxt, pos)
    return pos


def _sc_segment_sum(n_rows):
    mesh = plsc.VectorSubcoreMesh(core_axis_name="c", subcore_axis_name="s")
    cp = pltpu.CompilerParams()
    if "needs_layout_passes" in pltpu.CompilerParams.__dataclass_fields__:
        cp = dataclasses.replace(cp, needs_layout_passes=False)

    @functools.partial(
        pl.kernel,
        mesh=mesh,
        compiler_params=cp,
        out_type=(
            jax.ShapeDtypeStruct((NUM_CLASSES, FEAT), jnp.float32),
            jax.ShapeDtypeStruct((NUM_CLASSES, LANES), jnp.float32),
        ),
        scratch_types=[
            pltpu.VMEM((n_rows + LANES,), jnp.int32),
            pltpu.VMEM((CHUNK, FEAT), jnp.float32),
            pltpu.VMEM((CLS_PER_TILE, FEAT), jnp.float32),
            pltpu.VMEM((CLS_PER_TILE, LANES), jnp.float32),
        ],
    )
    def sc_kernel(sup_hbm, labels_hbm, sums_hbm, cnts_hbm,
                  labels_v, rows_v, acc_v, cnt_v):
        cid = lax.axis_index("c")
        sid = lax.axis_index("s")
        wid = cid * SC_SUBCORES + sid
        cls0 = wid * CLS_PER_TILE

        pltpu.sync_copy(labels_hbm, labels_v.at[pl.ds(0, n_rows)])

        col_iota = lax.iota(jnp.int32, LANES)
        cols = [col_iota + c * LANES for c in range(FEAT_CHUNKS)]
        zero_acc = tuple(jnp.zeros((LANES,), jnp.float32)
                         for _ in range(FEAT_CHUNKS))

        def class_body(cc, _):
            lo = _bsearch_ge(labels_v, cls0 + cc, n_rows)
            hi = _bsearch_ge(labels_v, cls0 + cc + 1, n_rows)
            alo = (lo // 8) * 8
            n_ch = lax.div(hi - alo + (CHUNK - 1), CHUNK)

            def chunk_body(j, accs):
                start = alo + j * CHUNK
                start_eff = jnp.minimum(start, n_rows - CHUNK)
                pltpu.sync_copy(sup_hbm.at[pl.ds(start_eff, CHUNK)], rows_v)

                def row_body(k, accs):
                    r = start_eff + k
                    cond = jnp.logical_and(
                        jnp.logical_and(r >= start, r >= lo), r < hi)
                    k_idx = jnp.full((LANES,), k, jnp.int32)
                    out = []
                    for c in range(FEAT_CHUNKS):
                        data = plsc.load_gather(rows_v, [k_idx, cols[c]])
                        data = jnp.where(cond, data, 0.0)
                        out.append(accs[c] + data)
                    return tuple(out)

                return lax.fori_loop(0, 1, row_body, accs)

            accs = lax.fori_loop(0, n_ch, chunk_body, zero_acc)
            cc_idx = jnp.full((LANES,), cc, jnp.int32)
            for c in range(FEAT_CHUNKS):
                plsc.store_scatter(acc_v, [cc_idx, cols[c]], accs[c])
            cntv = jnp.full((LANES,), 1.0, jnp.float32) * (hi - lo).astype(
                jnp.float32)
            plsc.store_scatter(cnt_v, [cc_idx, col_iota], cntv)
            return 0

        lax.fori_loop(0, CLS_PER_TILE, class_body, 0)

        pltpu.sync_copy(acc_v, sums_hbm.at[pl.ds(cls0, CLS_PER_TILE)])
        pltpu.sync_copy(cnt_v, cnts_hbm.at[pl.ds(cls0, CLS_PER_TILE)])

    return sc_kernel


def _dist_kernel(sums_ref, cnts_ref, q_ref, out_ref, ptT_ref, p2_ref):
    i = pl.program_id(0)

    @pl.when(i == 0)
    def _finalize_protos():
        cnt = cnts_ref[:, :1]
        protos = sums_ref[...] / jnp.maximum(cnt, 1.0)
        ptT = protos.T
        p2_ref[...] = jnp.sum(ptT * ptT, axis=0, keepdims=True)
        ptT_ref[...] = ptT.astype(jnp.bfloat16)

    qb = q_ref[...]
    q2 = jnp.sum(qb * qb, axis=1, keepdims=True)
    cross = jax.lax.dot_general(
        qb.astype(jnp.bfloat16), ptT_ref[...],
        dimension_numbers=(((1,), (0,)), ((), ())),
        preferred_element_type=jnp.float32)
    d2 = (q2 + p2_ref[...]) - 2.0 * cross
    out_ref[...] = -jnp.sqrt(jnp.maximum(d2, 0.0))


@jax.jit
def kernel(support_features, support_labels, query_features):
    n_sup = support_features.shape[0]
    n_q = query_features.shape[0]
    labels_i32 = support_labels.astype(jnp.int32)

    sums, cnts = _sc_segment_sum(n_sup)(support_features, labels_i32)

    out = pl.pallas_call(
        _dist_kernel,
        grid=(n_q // Q_BLOCK,),
        in_specs=[
            pl.BlockSpec((NUM_CLASSES, FEAT), lambda i: (0, 0)),
            pl.BlockSpec((NUM_CLASSES, LANES), lambda i: (0, 0)),
            pl.BlockSpec((Q_BLOCK, FEAT), lambda i: (i, 0)),
        ],
        out_specs=pl.BlockSpec((Q_BLOCK, NUM_CLASSES), lambda i: (i, 0)),
        out_shape=jax.ShapeDtypeStruct((n_q, NUM_CLASSES), jnp.float32),
        scratch_shapes=[
            pltpu.VMEM((FEAT, NUM_CLASSES), jnp.bfloat16),
            pltpu.VMEM((1, NUM_CLASSES), jnp.float32),
        ],
    )(sums, cnts, query_features)
    return out


# speedup vs baseline: 2.2421x; 2.2421x over previous
import jax
import jax.numpy as jnp
from jax.experimental import pallas as pl
from jax.experimental.pallas import tpu as pltpu

NUM_CLASSES = 256
FEAT = 512
SUP_BLOCK = 4096
Q_BLOCK = 4096


def _proto_kernel(labels_ref, sup_ref, out_ref, acc_ref, cnt_ref):
    i = pl.program_id(0)
    nsteps = pl.num_programs(0)
    labels = labels_ref[i]
    classes = jax.lax.broadcasted_iota(jnp.int32, (NUM_CLASSES, SUP_BLOCK), 0)
    onehot = (classes == labels[None, :]).astype(jnp.float32)
    sb = sup_ref[...].astype(jnp.bfloat16)
    partial = jax.lax.dot_general(
        onehot.astype(jnp.bfloat16), sb,
        dimension_numbers=(((1,), (0,)), ((), ())),
        preferred_element_type=jnp.float32)
    pcnt = jnp.sum(onehot, axis=1, keepdims=True)

    @pl.when(i == 0)
    def _init():
        acc_ref[...] = partial
        cnt_ref[...] = pcnt

    @pl.when(i > 0)
    def _acc():
        acc_ref[...] += partial
        cnt_ref[...] += pcnt

    @pl.when(i == nsteps - 1)
    def _finalize():
        protos = acc_ref[...] / jnp.maximum(cnt_ref[...], 1.0)
        out_ref[...] = protos.T


def _dist_kernel(q_ref, pt_ref, out_ref):
    pt = pt_ref[...]
    p2 = jnp.sum(pt * pt, axis=0, keepdims=True)
    qb = q_ref[...]
    q2 = jnp.sum(qb * qb, axis=1, keepdims=True)
    cross = jax.lax.dot_general(
        qb.astype(jnp.bfloat16), pt.astype(jnp.bfloat16),
        dimension_numbers=(((1,), (0,)), ((), ())),
        preferred_element_type=jnp.float32)
    d2 = (q2 + p2) - 2.0 * cross
    out_ref[...] = -jnp.sqrt(jnp.maximum(d2, 0.0))


@jax.jit
def kernel(support_features, support_labels, query_features):
    n_sup = support_features.shape[0]
    n_q = query_features.shape[0]
    labels2d = support_labels.astype(jnp.int32).reshape(
        n_sup // SUP_BLOCK, SUP_BLOCK)

    protoT = pl.pallas_call(
        _proto_kernel,
        grid=(n_sup // SUP_BLOCK,),
        in_specs=[
            pl.BlockSpec(labels2d.shape, lambda i: (0, 0)),
            pl.BlockSpec((SUP_BLOCK, FEAT), lambda i: (i, 0)),
        ],
        out_specs=pl.BlockSpec((FEAT, NUM_CLASSES), lambda i: (0, 0)),
        out_shape=jax.ShapeDtypeStruct((FEAT, NUM_CLASSES), jnp.float32),
        scratch_shapes=[
            pltpu.VMEM((NUM_CLASSES, FEAT), jnp.float32),
            pltpu.VMEM((NUM_CLASSES, 1), jnp.float32),
        ],
    )(labels2d, support_features)

    out = pl.pallas_call(
        _dist_kernel,
        grid=(n_q // Q_BLOCK,),
        in_specs=[
            pl.BlockSpec((Q_BLOCK, FEAT), lambda i: (i, 0)),
            pl.BlockSpec((FEAT, NUM_CLASSES), lambda i: (0, 0)),
        ],
        out_specs=pl.BlockSpec((Q_BLOCK, NUM_CLASSES), lambda i: (i, 0)),
        out_shape=jax.ShapeDtypeStruct((n_q, NUM_CLASSES), jnp.float32),
    )(query_features, protoT)
    return out
